# baseline (speedup 1.0000x reference)
import dataclasses
import functools

import numpy as np

import jax
import jax.numpy as jnp
from jax import lax
from jax.experimental import pallas as pl
from jax.experimental.pallas import tpu as pltpu
from jax.experimental.pallas import tpu_sc as plsc

N = 10000
E = 320000
D = 128
NPAD = 10240
NC, NS, L = 2, 16, 16
NW = NC * NS
CH = 80
EPAD = NW * CH * 128
RPT = NPAD // NS


def _mesh():
    return plsc.VectorSubcoreMesh(core_axis_name="c", subcore_axis_name="s")


def _sc_params(**kw):
    cp = pltpu.CompilerParams(**kw)
    if "needs_layout_passes" in pltpu.CompilerParams.__dataclass_fields__:
        cp = dataclasses.replace(cp, needs_layout_passes=False)
    return cp


HIST_WIN = 8


def _sc_hist(dst2d):
    @functools.partial(
        pl.kernel,
        out_type=jax.ShapeDtypeStruct((NC, NPAD), jnp.float32),
        mesh=_mesh(),
        scratch_types=[
            pltpu.VMEM((CH, 128), jnp.int32),
            pltpu.VMEM((128,), jnp.float32),
            pltpu.VMEM((RPT,), jnp.float32),
            pltpu.VMEM_SHARED((NPAD,), jnp.float32),
            pltpu.SemaphoreType.DMA,
        ],
    )
    def k(dst_hbm, out_hbm, dst_v, ones_v, zv, hist_sh, sem):
        c = lax.axis_index("c")
        s = lax.axis_index("s")
        w = c * NS + s
        pltpu.sync_copy(dst_hbm.at[pl.ds(w * CH, CH)], dst_v)

        @pl.loop(0, 8)
        def _(i):
            ones_v[pl.ds(i * L, L)] = jnp.ones((L,), jnp.float32)

        @pl.loop(0, RPT // L)
        def _(i):
            zv[pl.ds(i * L, L)] = jnp.zeros((L,), jnp.float32)

        pltpu.sync_copy(zv, hist_sh.at[pl.ds(s * RPT, RPT)])
        plsc.subcore_barrier()

        @pl.loop(0, CH)
        def _(j):
            pltpu.async_copy(ones_v, hist_sh.at[dst_v.at[j]], sem, add=True)

            @pl.when(j >= HIST_WIN)
            def _():
                pltpu.make_async_copy(
                    ones_v, hist_sh.at[dst_v.at[j - HIST_WIN]], sem).wait()

        @pl.loop(CH - HIST_WIN, CH)
        def _(j):
            pltpu.make_async_copy(ones_v, hist_sh.at[dst_v.at[j]], sem).wait()

        plsc.subcore_barrier()
        pltpu.sync_copy(hist_sh.at[pl.ds(s * RPT, RPT)],
                        out_hbm.at[c, pl.ds(s * RPT, RPT)])

    return k(dst2d)


NQ = 5
QC = CH // NQ


def _sc_msg(ybf, src2d, dst2d):
    @functools.partial(
        pl.kernel,
        out_type=jax.ShapeDtypeStruct((NC, NPAD, D), jnp.float32),
        mesh=_mesh(),
        scratch_types=[
            [pltpu.VMEM((QC, 128), jnp.int32)] * 2,
            [pltpu.VMEM((2 * QC, 64), jnp.int32)] * 2,
            [pltpu.VMEM((128, D // 2), jnp.int32)] * 2,
            [pltpu.VMEM((64, D), jnp.float32)] * 2,
            pltpu.VMEM_SHARED((NPAD, D), jnp.float32),
            [pltpu.SemaphoreType.DMA] * 2,
            [pltpu.SemaphoreType.DMA] * 2,
            pltpu.SemaphoreType.DMA,
        ],
        compiler_params=_sc_params(use_tc_tiling_on_sc=False),
    )
    def k(y_hbm, src_hbm, dst_hbm, out_hbm, srcq, dstq, bfg, fbuf, accum,
          gsems, ssems, isem):
        c = lax.axis_index("c")
        s = lax.axis_index("s")
        w = c * NS + s
        base = w * CH

        pltpu.async_copy(src_hbm.at[pl.ds(base, QC)], srcq[0], isem)
        pltpu.async_copy(dst_hbm.at[pl.ds(2 * base, 2 * QC)], dstq[0], isem)

        @pl.loop(0, 64)
        def _(i):
            @pl.loop(0, D // L)
            def _(q):
                fbuf[0][i, pl.ds(q * L, L)] = jnp.zeros((L,), jnp.float32)

        for r in range(RPT // 64):
            pltpu.sync_copy(fbuf[0], accum.at[pl.ds(s * RPT + r * 64, 64)])

        pltpu.make_async_copy(src_hbm.at[pl.ds(base, QC)], srcq[0],
                              isem).wait()
        pltpu.make_async_copy(dst_hbm.at[pl.ds(2 * base, 2 * QC)], dstq[0],
                              isem).wait()
        plsc.subcore_barrier()

        def g_start(i, sl, b):
            pltpu.async_copy(y_hbm.at[srcq[sl].at[i]], bfg[b], gsems[b])

        def g_wait(i, sl, b):
            pltpu.make_async_copy(y_hbm.at[srcq[sl].at[i]], bfg[b],
                                  gsems[b]).wait()

        def s_start(i, sl, gb, h):
            pltpu.async_copy(fbuf[h], accum.at[dstq[sl].at[2 * i + h]],
                             ssems[h], add=True)

        def s_wait(i, sl, h):
            pltpu.make_async_copy(fbuf[h], accum.at[dstq[sl].at[2 * i + h]],
                                  ssems[h]).wait()

        def convert(i, sl, gb, h, first):
            if first:
                @pl.when(i > 0)
                def _():
                    s_wait(i, sl, h)
            else:
                s_wait(i, sl, h)

            @pl.loop(0, 64)
            def _(r):
                for g4 in range(D // 32):
                    v = bfg[gb][h * 64 + r, pl.ds(L * g4, L)]
                    a = plsc.bitcast(v << 16, jnp.float32)
                    b2 = plsc.bitcast(v & jnp.int32(-65536), jnp.float32)
                    fbuf[h][r, pl.ds(32 * g4, L)] = a
                    fbuf[h][r, pl.ds(32 * g4 + L, L)] = b2

            s_start(i, sl, gb, h)

        for q in range(NQ):
            sl = q % 2
            if q < NQ - 1:
                nb = base + (q + 1) * QC
                pltpu.async_copy(src_hbm.at[pl.ds(nb, QC)], srcq[1 - sl],
                                 isem)
                pltpu.async_copy(dst_hbm.at[pl.ds(2 * nb, 2 * QC)],
                                 dstq[1 - sl], isem)

            for gb in range(2):
                g_start(gb, sl, gb)

            @pl.loop(0, QC - 2, step=2)
            def _(i):
                for gb in range(2):
                    ii = i + gb
                    g_wait(ii, sl, gb)
                    for h in range(2):
                        convert(ii, sl, gb, h, first=(q == 0))
                    g_start(ii + 2, sl, gb)

            for gb in range(2):
                ii = QC - 2 + gb
                g_wait(ii, sl, gb)
                for h in range(2):
                    convert(ii, sl, gb, h, first=False)

            if q < NQ - 1:
                nb = base + (q + 1) * QC
                pltpu.make_async_copy(src_hbm.at[pl.ds(nb, QC)],
                                      srcq[1 - sl], isem).wait()
                pltpu.make_async_copy(dst_hbm.at[pl.ds(2 * nb, 2 * QC)],
                                      dstq[1 - sl], isem).wait()

        for h in range(2):
            s_wait(QC - 1, (NQ - 1) % 2, h)

        plsc.subcore_barrier()
        pltpu.sync_copy(accum.at[pl.ds(s * RPT, RPT)],
                        out_hbm.at[c, pl.ds(s * RPT, RPT)])

    return k(ybf, src2d, dst2d)


def _tc_mm(x_pad, w):
    def body(x_ref, w_ref, xw_ref):
        xw_ref[...] = jnp.dot(x_ref[...], w_ref[...],
                              preferred_element_type=jnp.float32,
                              precision=lax.Precision.HIGHEST)

    return pl.pallas_call(
        body,
        out_shape=jax.ShapeDtypeStruct((NPAD, D), jnp.float32),
    )(x_pad, w)


def _tc_scale(xw, hist_t, perm):
    def body(xw_ref, h_ref, p_ref, y_ref):
        deg = h_ref[:, 0:1] + h_ref[:, 1:2] + 1.0
        xwp = jnp.dot(xw_ref[...], p_ref[...],
                      preferred_element_type=jnp.float32,
                      precision=lax.Precision.HIGHEST)
        y_ref[...] = (xwp * lax.rsqrt(deg)).astype(jnp.bfloat16)

    return pl.pallas_call(
        body,
        out_shape=jax.ShapeDtypeStruct((NPAD, D), jnp.bfloat16),
    )(xw, hist_t, perm)


def _tc_final(partials, hist_t, xw, b2d):
    def body(p_ref, h_ref, xw_ref, b_ref, o_ref):
        deg = h_ref[:, 0:1] + h_ref[:, 1:2] + 1.0
        dis = lax.rsqrt(deg)
        agg = p_ref[0] + p_ref[1]
        res = agg * dis + xw_ref[...] * (dis * dis) + b_ref[...]
        o_ref[...] = res[:N, :]

    return pl.pallas_call(
        body,
        out_shape=jax.ShapeDtypeStruct((N, D), jnp.float32),
    )(partials, hist_t, xw, b2d)


_M2O = np.arange(D).reshape(D // 32, 2, 16).transpose(0, 2, 1).reshape(D)
_PERM = np.eye(D, dtype=np.float32)[_M2O].T


def kernel(x, edge_index, W, b):
    src = edge_index[0]
    dst = edge_index[1]
    pad = EPAD - E
    pad_idx = N + (jnp.arange(pad, dtype=jnp.int32) % (NPAD - N))
    src_p = jnp.concatenate([src, pad_idx]).reshape(NW * CH, 128)
    dst_p = jnp.concatenate([dst, pad_idx]).reshape(NW * CH, 128)
    dst_p64 = dst_p.reshape(NW * CH * 2, 64)
    x_pad = jnp.pad(x, ((0, NPAD - N), (0, 0)))

    hist = _sc_hist(dst_p)
    xw = _tc_mm(x_pad, W)
    hist_t = hist.T
    y = _tc_scale(xw, hist_t, jnp.asarray(_PERM))
    y32 = lax.bitcast_convert_type(y.reshape(NPAD, D // 2, 2), jnp.int32)
    part = _sc_msg(y32, src_p, dst_p64)
    return _tc_final(part, hist_t, xw, b.reshape(1, D))


# speedup vs baseline: 1.4715x; 1.4715x over previous
import dataclasses
import functools

import numpy as np

import jax
import jax.numpy as jnp
from jax import lax
from jax.experimental import pallas as pl
from jax.experimental.pallas import tpu as pltpu
from jax.experimental.pallas import tpu_sc as plsc

N = 10000
E = 320000
D = 128
NPAD = 10240
NC, NS, L = 2, 16, 16
NW = NC * NS
CW = 125
CH = 80
RPT = NPAD // NS


def _mesh():
    return plsc.VectorSubcoreMesh(core_axis_name="c", subcore_axis_name="s")


def _sc_params(**kw):
    cp = pltpu.CompilerParams(**kw)
    if "needs_layout_passes" in pltpu.CompilerParams.__dataclass_fields__:
        cp = dataclasses.replace(cp, needs_layout_passes=False)
    return cp


HIST_WIN = 8


def _sc_hist(dst2d):
    @functools.partial(
        pl.kernel,
        out_type=jax.ShapeDtypeStruct((NC, NPAD), jnp.float32),
        mesh=_mesh(),
        scratch_types=[
            pltpu.VMEM((CH, CW), jnp.int32),
            pltpu.VMEM((128,), jnp.float32),
            pltpu.VMEM((RPT,), jnp.float32),
            pltpu.VMEM_SHARED((NPAD,), jnp.float32),
            pltpu.SemaphoreType.DMA,
        ],
    )
    def k(dst_hbm, out_hbm, dst_v, ones_v, zv, hist_sh, sem):
        c = lax.axis_index("c")
        s = lax.axis_index("s")
        w = c * NS + s
        pltpu.sync_copy(dst_hbm.at[pl.ds(w * CH, CH)], dst_v)

        @pl.loop(0, 8)
        def _(i):
            ones_v[pl.ds(i * L, L)] = jnp.ones((L,), jnp.float32)

        @pl.loop(0, RPT // L)
        def _(i):
            zv[pl.ds(i * L, L)] = jnp.zeros((L,), jnp.float32)

        pltpu.sync_copy(zv, hist_sh.at[pl.ds(s * RPT, RPT)])
        plsc.subcore_barrier()

        ones_s = ones_v.at[pl.ds(0, CW)]

        @pl.loop(0, CH)
        def _(j):
            pltpu.async_copy(ones_s, hist_sh.at[dst_v.at[j]], sem, add=True)

            @pl.when(j >= HIST_WIN)
            def _():
                pltpu.make_async_copy(
                    ones_s, hist_sh.at[dst_v.at[j - HIST_WIN]], sem).wait()

        @pl.loop(CH - HIST_WIN, CH)
        def _(j):
            pltpu.make_async_copy(ones_s, hist_sh.at[dst_v.at[j]], sem).wait()

        plsc.subcore_barrier()
        pltpu.sync_copy(hist_sh.at[pl.ds(s * RPT, RPT)],
                        out_hbm.at[c, pl.ds(s * RPT, RPT)])

    return k(dst2d)


NQ = 5
QC = CH // NQ


def _sc_msg(y, src2d, dst2d):
    @functools.partial(
        pl.kernel,
        out_type=jax.ShapeDtypeStruct((NC, NPAD, D), jnp.float32),
        mesh=_mesh(),
        scratch_types=[
            [pltpu.VMEM((QC, CW), jnp.int32)] * 2,
            [pltpu.VMEM((QC, CW), jnp.int32)] * 2,
            [pltpu.VMEM((CW, D), jnp.float32)] * 2,
            pltpu.VMEM((32, D), jnp.float32),
            pltpu.VMEM_SHARED((NPAD, D), jnp.float32),
            [pltpu.SemaphoreType.DMA] * 2,
            [pltpu.SemaphoreType.DMA] * 2,
            pltpu.SemaphoreType.DMA,
        ],
    )
    def k(y_hbm, src_hbm, dst_hbm, out_hbm, srcq, dstq, bufs, zb, accum,
          gsems, ssems, isem):
        c = lax.axis_index("c")
        s = lax.axis_index("s")
        w = c * NS + s
        base = w * CH

        pltpu.async_copy(src_hbm.at[pl.ds(base, QC)], srcq[0], isem)
        pltpu.async_copy(dst_hbm.at[pl.ds(base, QC)], dstq[0], isem)

        @pl.loop(0, 32)
        def _(i):
            @pl.loop(0, D // L)
            def _(q):
                zb[i, pl.ds(q * L, L)] = jnp.zeros((L,), jnp.float32)

        for r in range(RPT // 32):
            pltpu.sync_copy(zb, accum.at[pl.ds(s * RPT + r * 32, 32)])

        pltpu.make_async_copy(src_hbm.at[pl.ds(base, QC)], srcq[0],
                              isem).wait()
        pltpu.make_async_copy(dst_hbm.at[pl.ds(base, QC)], dstq[0],
                              isem).wait()
        plsc.subcore_barrier()

        def g_start(i, sl, b):
            pltpu.async_copy(y_hbm.at[srcq[sl].at[i]], bufs[b], gsems[b])

        def g_wait(i, sl, b):
            pltpu.make_async_copy(y_hbm.at[srcq[sl].at[i]], bufs[b],
                                  gsems[b]).wait()

        def s_start(i, sl, b):
            pltpu.async_copy(bufs[b], accum.at[dstq[sl].at[i]], ssems[b],
                             add=True)

        def s_wait(i, sl, b):
            pltpu.make_async_copy(bufs[b], accum.at[dstq[sl].at[i]],
                                  ssems[b]).wait()

        for q in range(NQ):
            sl = q % 2
            if q < NQ - 1:
                nb = base + (q + 1) * QC
                pltpu.async_copy(src_hbm.at[pl.ds(nb, QC)], srcq[1 - sl],
                                 isem)
                pltpu.async_copy(dst_hbm.at[pl.ds(nb, QC)], dstq[1 - sl],
                                 isem)

            for b in range(2):
                g_start(b, sl, b)

            @pl.loop(0, QC - 2, step=2)
            def _(i):
                g_wait(i, sl, 0)
                s_start(i, sl, 0)
                g_wait(i + 1, sl, 1)
                s_start(i + 1, sl, 1)
                s_wait(i, sl, 0)
                g_start(i + 2, sl, 0)
                s_wait(i + 1, sl, 1)
                g_start(i + 3, sl, 1)

            g_wait(QC - 2, sl, 0)
            s_start(QC - 2, sl, 0)
            g_wait(QC - 1, sl, 1)
            s_start(QC - 1, sl, 1)
            s_wait(QC - 2, sl, 0)
            s_wait(QC - 1, sl, 1)

            if q < NQ - 1:
                nb = base + (q + 1) * QC
                pltpu.make_async_copy(src_hbm.at[pl.ds(nb, QC)],
                                      srcq[1 - sl], isem).wait()
                pltpu.make_async_copy(dst_hbm.at[pl.ds(nb, QC)],
                                      dstq[1 - sl], isem).wait()

        plsc.subcore_barrier()
        pltpu.sync_copy(accum.at[pl.ds(s * RPT, RPT)],
                        out_hbm.at[c, pl.ds(s * RPT, RPT)])

    return k(y, src2d, dst2d)


def _tc_mm(x_pad, w, hist_t):
    def body(x_ref, w_ref, h_ref, y_ref, xw_ref):
        deg = h_ref[:, 0:1] + h_ref[:, 1:2] + 1.0
        dis = lax.rsqrt(deg)
        xw = jnp.dot(x_ref[...], w_ref[...],
                     preferred_element_type=jnp.float32,
                     precision=lax.Precision.HIGHEST)
        xw_ref[...] = xw
        y_ref[...] = xw * dis

    return pl.pallas_call(
        body,
        out_shape=(jax.ShapeDtypeStruct((NPAD, D), jnp.float32),
                   jax.ShapeDtypeStruct((NPAD, D), jnp.float32)),
    )(x_pad, w, hist_t)


def _tc_final(partials, hist_t, xw, b2d):
    def body(p_ref, h_ref, xw_ref, b_ref, o_ref):
        deg = h_ref[:, 0:1] + h_ref[:, 1:2] + 1.0
        dis = lax.rsqrt(deg)
        agg = p_ref[0] + p_ref[1]
        res = agg * dis + xw_ref[...] * (dis * dis) + b_ref[...]
        o_ref[...] = res[:N, :]

    return pl.pallas_call(
        body,
        out_shape=jax.ShapeDtypeStruct((N, D), jnp.float32),
    )(partials, hist_t, xw, b2d)


def kernel(x, edge_index, W, b):
    src_p = edge_index[0].reshape(NW * CH, CW)
    dst_p = edge_index[1].reshape(NW * CH, CW)
    x_pad = jnp.pad(x, ((0, NPAD - N), (0, 0)))

    hist = _sc_hist(dst_p)
    hist_t = hist.T
    y, xw = _tc_mm(x_pad, W, hist_t)
    part = _sc_msg(y, src_p, dst_p)
    return _tc_final(part, hist_t, xw, b.reshape(1, D))


# speedup vs baseline: 1.7286x; 1.1747x over previous
import dataclasses
import functools

import numpy as np

import jax
import jax.numpy as jnp
from jax import lax
from jax.experimental import pallas as pl
from jax.experimental.pallas import tpu as pltpu
from jax.experimental.pallas import tpu_sc as plsc

N = 10000
E = 320000
D = 128
NPAD = 10240
NC, NS, L = 2, 16, 16
NW = NC * NS
CW = 125
CH = 80
RPT = NPAD // NS


def _mesh():
    return plsc.VectorSubcoreMesh(core_axis_name="c", subcore_axis_name="s")


def _sc_params(**kw):
    cp = pltpu.CompilerParams(**kw)
    if "needs_layout_passes" in pltpu.CompilerParams.__dataclass_fields__:
        cp = dataclasses.replace(cp, needs_layout_passes=False)
    return cp


HIST_WIN = 8


def _sc_hist(dst2d):
    @functools.partial(
        pl.kernel,
        out_type=jax.ShapeDtypeStruct((NC, NPAD), jnp.float32),
        mesh=_mesh(),
        scratch_types=[
            pltpu.VMEM((CH, CW), jnp.int32),
            pltpu.VMEM((128,), jnp.float32),
            pltpu.VMEM((RPT,), jnp.float32),
            pltpu.VMEM_SHARED((NPAD,), jnp.float32),
            pltpu.SemaphoreType.DMA,
        ],
    )
    def k(dst_hbm, out_hbm, dst_v, ones_v, zv, hist_sh, sem):
        c = lax.axis_index("c")
        s = lax.axis_index("s")
        w = c * NS + s
        pltpu.sync_copy(dst_hbm.at[pl.ds(w * CH, CH)], dst_v)

        @pl.loop(0, 8)
        def _(i):
            ones_v[pl.ds(i * L, L)] = jnp.ones((L,), jnp.float32)

        @pl.loop(0, RPT // L)
        def _(i):
            zv[pl.ds(i * L, L)] = jnp.zeros((L,), jnp.float32)

        pltpu.sync_copy(zv, hist_sh.at[pl.ds(s * RPT, RPT)])
        plsc.subcore_barrier()

        ones_s = ones_v.at[pl.ds(0, CW)]

        @pl.loop(0, CH)
        def _(j):
            pltpu.async_copy(ones_s, hist_sh.at[dst_v.at[j]], sem, add=True)

            @pl.when(j >= HIST_WIN)
            def _():
                pltpu.make_async_copy(
                    ones_s, hist_sh.at[dst_v.at[j - HIST_WIN]], sem).wait()

        @pl.loop(CH - HIST_WIN, CH)
        def _(j):
            pltpu.make_async_copy(ones_s, hist_sh.at[dst_v.at[j]], sem).wait()

        plsc.subcore_barrier()
        pltpu.sync_copy(hist_sh.at[pl.ds(s * RPT, RPT)],
                        out_hbm.at[c, pl.ds(s * RPT, RPT)])

    return k(dst2d)


NQ = 5
QC = CH // NQ


def _sc_msg(y, src2d, dst2d):
    @functools.partial(
        pl.kernel,
        out_type=jax.ShapeDtypeStruct((NC, NPAD, D), jnp.float32),
        mesh=_mesh(),
        scratch_types=[
            [pltpu.VMEM((QC, CW), jnp.int32)] * 2,
            [pltpu.VMEM((QC, CW), jnp.int32)] * 2,
            [pltpu.VMEM((CW, D), jnp.float32)] * 2,
            pltpu.VMEM((32, D), jnp.float32),
            pltpu.VMEM_SHARED((NPAD, D), jnp.float32),
            [pltpu.SemaphoreType.DMA] * 2,
            [pltpu.SemaphoreType.DMA] * 2,
            pltpu.SemaphoreType.DMA,
        ],
    )
    def k(y_hbm, src_hbm, dst_hbm, out_hbm, srcq, dstq, bufs, zb, accum,
          gsems, ssems, isem):
        c = lax.axis_index("c")
        s = lax.axis_index("s")
        w = c * NS + s
        base = w * CH

        pltpu.async_copy(src_hbm.at[pl.ds(base, QC)], srcq[0], isem)
        pltpu.async_copy(dst_hbm.at[pl.ds(base, QC)], dstq[0], isem)

        @pl.loop(0, 32)
        def _(i):
            @pl.loop(0, D // L)
            def _(q):
                zb[i, pl.ds(q * L, L)] = jnp.zeros((L,), jnp.float32)

        for r in range(RPT // 32):
            pltpu.sync_copy(zb, accum.at[pl.ds(s * RPT + r * 32, 32)])

        pltpu.make_async_copy(src_hbm.at[pl.ds(base, QC)], srcq[0],
                              isem).wait()
        pltpu.make_async_copy(dst_hbm.at[pl.ds(base, QC)], dstq[0],
                              isem).wait()
        plsc.subcore_barrier()

        def g_start(i, sl, b):
            pltpu.async_copy(y_hbm.at[srcq[sl].at[i]], bufs[b], gsems[b])

        def g_wait(i, sl, b):
            pltpu.make_async_copy(y_hbm.at[srcq[sl].at[i]], bufs[b],
                                  gsems[b]).wait()

        def s_start(i, sl, b):
            pltpu.async_copy(bufs[b], accum.at[dstq[sl].at[i]], ssems[b],
                             add=True)

        def s_wait(i, sl, b):
            pltpu.make_async_copy(bufs[b], accum.at[dstq[sl].at[i]],
                                  ssems[b]).wait()

        for q in range(NQ):
            sl = q % 2
            if q < NQ - 1:
                nb = base + (q + 1) * QC
                pltpu.async_copy(src_hbm.at[pl.ds(nb, QC)], srcq[1 - sl],
                                 isem)
                pltpu.async_copy(dst_hbm.at[pl.ds(nb, QC)], dstq[1 - sl],
                                 isem)

            for b in range(2):
                g_start(b, sl, b)

            @pl.loop(0, QC - 2, step=2)
            def _(i):
                for b in range(2):
                    ii = i + b
                    g_wait(ii, sl, b)
                    pltpu.sync_copy(bufs[b], accum.at[dstq[sl].at[ii]],
                                    add=True)
                    g_start(ii + 2, sl, b)

            for b in range(2):
                ii = QC - 2 + b
                g_wait(ii, sl, b)
                pltpu.sync_copy(bufs[b], accum.at[dstq[sl].at[ii]], add=True)

            if q < NQ - 1:
                nb = base + (q + 1) * QC
                pltpu.make_async_copy(src_hbm.at[pl.ds(nb, QC)],
                                      srcq[1 - sl], isem).wait()
                pltpu.make_async_copy(dst_hbm.at[pl.ds(nb, QC)],
                                      dstq[1 - sl], isem).wait()

        plsc.subcore_barrier()
        pltpu.sync_copy(accum.at[pl.ds(s * RPT, RPT)],
                        out_hbm.at[c, pl.ds(s * RPT, RPT)])

    return k(y, src2d, dst2d)


def _tc_mm(x_pad, w, hist_t):
    def body(x_ref, w_ref, h_ref, y_ref, xw_ref):
        deg = h_ref[:, 0:1] + h_ref[:, 1:2] + 1.0
        dis = lax.rsqrt(deg)
        xw = jnp.dot(x_ref[...], w_ref[...],
                     preferred_element_type=jnp.float32,
                     precision=lax.Precision.HIGHEST)
        xw_ref[...] = xw
        y_ref[...] = xw * dis

    return pl.pallas_call(
        body,
        out_shape=(jax.ShapeDtypeStruct((NPAD, D), jnp.float32),
                   jax.ShapeDtypeStruct((NPAD, D), jnp.float32)),
    )(x_pad, w, hist_t)


def _tc_final(partials, hist_t, xw, b2d):
    def body(p_ref, h_ref, xw_ref, b_ref, o_ref):
        deg = h_ref[:, 0:1] + h_ref[:, 1:2] + 1.0
        dis = lax.rsqrt(deg)
        agg = p_ref[0] + p_ref[1]
        res = agg * dis + xw_ref[...] * (dis * dis) + b_ref[...]
        o_ref[...] = res[:N, :]

    return pl.pallas_call(
        body,
        out_shape=jax.ShapeDtypeStruct((N, D), jnp.float32),
    )(partials, hist_t, xw, b2d)


def kernel(x, edge_index, W, b):
    src_p = edge_index[0].reshape(NW * CH, CW)
    dst_p = edge_index[1].reshape(NW * CH, CW)
    x_pad = jnp.pad(x, ((0, NPAD - N), (0, 0)))

    hist = _sc_hist(dst_p)
    hist_t = hist.T
    y, xw = _tc_mm(x_pad, W, hist_t)
    part = _sc_msg(y, src_p, dst_p)
    return _tc_final(part, hist_t, xw, b.reshape(1, D))


# speedup vs baseline: 1.7530x; 1.0142x over previous
import dataclasses
import functools

import numpy as np

import jax
import jax.numpy as jnp
from jax import lax
from jax.experimental import pallas as pl
from jax.experimental.pallas import tpu as pltpu
from jax.experimental.pallas import tpu_sc as plsc

N = 10000
E = 320000
D = 128
NPAD = 10240
NC, NS, L = 2, 16, 16
NW = NC * NS
CW = 125
CH = 80
RPT = NPAD // NS


def _mesh():
    return plsc.VectorSubcoreMesh(core_axis_name="c", subcore_axis_name="s")


def _sc_params(**kw):
    cp = pltpu.CompilerParams(**kw)
    if "needs_layout_passes" in pltpu.CompilerParams.__dataclass_fields__:
        cp = dataclasses.replace(cp, needs_layout_passes=False)
    return cp


HIST_WIN = 8


def _sc_hist(dst2d):
    @functools.partial(
        pl.kernel,
        out_type=jax.ShapeDtypeStruct((NC, NPAD), jnp.float32),
        mesh=_mesh(),
        scratch_types=[
            pltpu.VMEM((CH, CW), jnp.int32),
            pltpu.VMEM((128,), jnp.float32),
            pltpu.VMEM((RPT,), jnp.float32),
            pltpu.VMEM_SHARED((NPAD,), jnp.float32),
            pltpu.SemaphoreType.DMA,
        ],
    )
    def k(dst_hbm, out_hbm, dst_v, ones_v, zv, hist_sh, sem):
        c = lax.axis_index("c")
        s = lax.axis_index("s")
        w = c * NS + s
        pltpu.sync_copy(dst_hbm.at[pl.ds(w * CH, CH)], dst_v)

        @pl.loop(0, 8)
        def _(i):
            ones_v[pl.ds(i * L, L)] = jnp.ones((L,), jnp.float32)

        @pl.loop(0, RPT // L)
        def _(i):
            zv[pl.ds(i * L, L)] = jnp.zeros((L,), jnp.float32)

        pltpu.sync_copy(zv, hist_sh.at[pl.ds(s * RPT, RPT)])
        plsc.subcore_barrier()

        ones_s = ones_v.at[pl.ds(0, CW)]

        @pl.loop(0, CH)
        def _(j):
            pltpu.async_copy(ones_s, hist_sh.at[dst_v.at[j]], sem, add=True)

            @pl.when(j >= HIST_WIN)
            def _():
                pltpu.make_async_copy(
                    ones_s, hist_sh.at[dst_v.at[j - HIST_WIN]], sem).wait()

        @pl.loop(CH - HIST_WIN, CH)
        def _(j):
            pltpu.make_async_copy(ones_s, hist_sh.at[dst_v.at[j]], sem).wait()

        plsc.subcore_barrier()
        pltpu.sync_copy(hist_sh.at[pl.ds(s * RPT, RPT)],
                        out_hbm.at[c, pl.ds(s * RPT, RPT)])

    return k(dst2d)


NQ = 5
QC = CH // NQ


def _sc_msg(y, src2d, dst2d):
    @functools.partial(
        pl.kernel,
        out_type=jax.ShapeDtypeStruct((NC, NPAD, D), jnp.float32),
        mesh=_mesh(),
        scratch_types=[
            pltpu.VMEM((CH, CW), jnp.int32),
            [pltpu.VMEM((QC, CW), jnp.int32)] * 2,
            [pltpu.VMEM((CW, D), jnp.float32)] * 2,
            pltpu.VMEM((8, D), jnp.float32),
            pltpu.VMEM_SHARED((NPAD, D), jnp.float32),
            [pltpu.SemaphoreType.DMA] * 2,
            pltpu.SemaphoreType.DMA,
        ],
    )
    def k(y_hbm, src_hbm, dst_hbm, out_hbm, src_v, dstq, bufs, zb, accum,
          gsems, isem):
        c = lax.axis_index("c")
        s = lax.axis_index("s")
        w = c * NS + s
        base = w * CH

        pltpu.async_copy(src_hbm.at[pl.ds(base, CH)], src_v, isem)
        pltpu.async_copy(dst_hbm.at[pl.ds(base, QC)], dstq[0], isem)

        @pl.loop(0, 8)
        def _(i):
            @pl.loop(0, D // L)
            def _(q):
                zb[i, pl.ds(q * L, L)] = jnp.zeros((L,), jnp.float32)

        for r in range(RPT // 8):
            pltpu.sync_copy(zb, accum.at[pl.ds(s * RPT + r * 8, 8)])

        pltpu.make_async_copy(src_hbm.at[pl.ds(base, CH)], src_v,
                              isem).wait()
        pltpu.make_async_copy(dst_hbm.at[pl.ds(base, QC)], dstq[0],
                              isem).wait()
        plsc.subcore_barrier()

        def g_start(i, b):
            pltpu.async_copy(y_hbm.at[src_v.at[i]], bufs[b], gsems[b])

        def g_wait(i, b):
            pltpu.make_async_copy(y_hbm.at[src_v.at[i]], bufs[b],
                                  gsems[b]).wait()

        g_start(0, 0)
        g_start(1, 1)
        for q in range(NQ):
            sl = q % 2
            q0 = q * QC
            if q < NQ - 1:
                pltpu.async_copy(dst_hbm.at[pl.ds(base + q0 + QC, QC)],
                                 dstq[1 - sl], isem)

            @pl.loop(q0, q0 + QC - 2, step=2)
            def _(i):
                for b in range(2):
                    ii = i + b
                    g_wait(ii, b)
                    pltpu.sync_copy(bufs[b],
                                    accum.at[dstq[sl].at[ii - q0]],
                                    add=True)
                    g_start(ii + 2, b)

            for b in range(2):
                ii = q0 + QC - 2 + b
                g_wait(ii, b)
                pltpu.sync_copy(bufs[b], accum.at[dstq[sl].at[ii - q0]],
                                add=True)
                if q < NQ - 1:
                    g_start(ii + 2, b)

            if q < NQ - 1:
                pltpu.make_async_copy(dst_hbm.at[pl.ds(base + q0 + QC, QC)],
                                      dstq[1 - sl], isem).wait()

        plsc.subcore_barrier()
        pltpu.sync_copy(accum.at[pl.ds(s * RPT, RPT)],
                        out_hbm.at[c, pl.ds(s * RPT, RPT)])

    return k(y, src2d, dst2d)


def _tc_mm(x_pad, w, hist_t):
    def body(x_ref, w_ref, h_ref, y_ref, xw_ref):
        deg = h_ref[:, 0:1] + h_ref[:, 1:2] + 1.0
        dis = lax.rsqrt(deg)
        xw = jnp.dot(x_ref[...], w_ref[...],
                     preferred_element_type=jnp.float32,
                     precision=lax.Precision.HIGHEST)
        xw_ref[...] = xw
        y_ref[...] = xw * dis

    return pl.pallas_call(
        body,
        out_shape=(jax.ShapeDtypeStruct((NPAD, D), jnp.float32),
                   jax.ShapeDtypeStruct((NPAD, D), jnp.float32)),
    )(x_pad, w, hist_t)


def _tc_final(partials, hist_t, xw, b2d):
    def body(p_ref, h_ref, xw_ref, b_ref, o_ref):
        deg = h_ref[:, 0:1] + h_ref[:, 1:2] + 1.0
        dis = lax.rsqrt(deg)
        agg = p_ref[0] + p_ref[1]
        res = agg * dis + xw_ref[...] * (dis * dis) + b_ref[...]
        o_ref[...] = res[:N, :]

    return pl.pallas_call(
        body,
        out_shape=jax.ShapeDtypeStruct((N, D), jnp.float32),
    )(partials, hist_t, xw, b2d)


def kernel(x, edge_index, W, b):
    src_p = edge_index[0].reshape(NW * CH, CW)
    dst_p = edge_index[1].reshape(NW * CH, CW)
    x_pad = jnp.pad(x, ((0, NPAD - N), (0, 0)))

    hist = _sc_hist(dst_p)
    hist_t = hist.T
    y, xw = _tc_mm(x_pad, W, hist_t)
    part = _sc_msg(y, src_p, dst_p)
    return _tc_final(part, hist_t, xw, b.reshape(1, D))


# speedup vs baseline: 1.7748x; 1.0124x over previous
import dataclasses
import functools

import numpy as np

import jax
import jax.numpy as jnp
from jax import lax
from jax.experimental import pallas as pl
from jax.experimental.pallas import tpu as pltpu
from jax.experimental.pallas import tpu_sc as plsc

N = 10000
E = 320000
D = 128
NPAD = 10240
NC, NS, L = 2, 16, 16
NW = NC * NS
CW = 125
CH = 80
RPT = NPAD // NS


def _mesh():
    return plsc.VectorSubcoreMesh(core_axis_name="c", subcore_axis_name="s")


def _sc_params(**kw):
    cp = pltpu.CompilerParams(**kw)
    if "needs_layout_passes" in pltpu.CompilerParams.__dataclass_fields__:
        cp = dataclasses.replace(cp, needs_layout_passes=False)
    return cp


HIST_WIN = 16


def _sc_hist(dst2d):
    @functools.partial(
        pl.kernel,
        out_type=jax.ShapeDtypeStruct((NC, NPAD), jnp.float32),
        mesh=_mesh(),
        scratch_types=[
            pltpu.VMEM((CH, CW), jnp.int32),
            pltpu.VMEM((128,), jnp.float32),
            pltpu.VMEM((RPT,), jnp.float32),
            pltpu.VMEM_SHARED((NPAD,), jnp.float32),
            pltpu.SemaphoreType.DMA,
        ],
    )
    def k(dst_hbm, out_hbm, dst_v, ones_v, zv, hist_sh, sem):
        c = lax.axis_index("c")
        s = lax.axis_index("s")
        w = c * NS + s
        pltpu.sync_copy(dst_hbm.at[pl.ds(w * CH, CH)], dst_v)

        @pl.loop(0, 8)
        def _(i):
            ones_v[pl.ds(i * L, L)] = jnp.ones((L,), jnp.float32)

        @pl.loop(0, RPT // L)
        def _(i):
            zv[pl.ds(i * L, L)] = jnp.zeros((L,), jnp.float32)

        pltpu.sync_copy(zv, hist_sh.at[pl.ds(s * RPT, RPT)])
        plsc.subcore_barrier()

        ones_s = ones_v.at[pl.ds(0, CW)]

        @pl.loop(0, CH)
        def _(j):
            pltpu.async_copy(ones_s, hist_sh.at[dst_v.at[j]], sem, add=True)

            @pl.when(j >= HIST_WIN)
            def _():
                pltpu.make_async_copy(
                    ones_s, hist_sh.at[dst_v.at[j - HIST_WIN]], sem).wait()

        @pl.loop(CH - HIST_WIN, CH)
        def _(j):
            pltpu.make_async_copy(ones_s, hist_sh.at[dst_v.at[j]], sem).wait()

        plsc.subcore_barrier()
        pltpu.sync_copy(hist_sh.at[pl.ds(s * RPT, RPT)],
                        out_hbm.at[c, pl.ds(s * RPT, RPT)])

    return k(dst2d)


NQ = 5
QC = CH // NQ


def _sc_msg(y, src2d, dst2d):
    @functools.partial(
        pl.kernel,
        out_type=jax.ShapeDtypeStruct((NC, NPAD, D), jnp.float32),
        mesh=_mesh(),
        scratch_types=[
            pltpu.VMEM((CH, CW), jnp.int32),
            [pltpu.VMEM((QC, CW), jnp.int32)] * 2,
            [pltpu.VMEM((CW, D), jnp.float32)] * 2,
            pltpu.VMEM((8, D), jnp.float32),
            pltpu.VMEM_SHARED((NPAD, D), jnp.float32),
            [pltpu.SemaphoreType.DMA] * 2,
            pltpu.SemaphoreType.DMA,
        ],
    )
    def k(y_hbm, src_hbm, dst_hbm, out_hbm, src_v, dstq, bufs, zb, accum,
          gsems, isem):
        c = lax.axis_index("c")
        s = lax.axis_index("s")
        w = c * NS + s
        base = w * CH

        pltpu.async_copy(src_hbm.at[pl.ds(base, CH)], src_v, isem)
        pltpu.async_copy(dst_hbm.at[pl.ds(base, QC)], dstq[0], isem)

        @pl.loop(0, 8)
        def _(i):
            @pl.loop(0, D // L)
            def _(q):
                zb[i, pl.ds(q * L, L)] = jnp.zeros((L,), jnp.float32)

        for r in range(RPT // 8):
            pltpu.sync_copy(zb, accum.at[pl.ds(s * RPT + r * 8, 8)])

        pltpu.make_async_copy(src_hbm.at[pl.ds(base, CH)], src_v,
                              isem).wait()
        pltpu.make_async_copy(dst_hbm.at[pl.ds(base, QC)], dstq[0],
                              isem).wait()
        plsc.subcore_barrier()

        def g_start(i, b):
            pltpu.async_copy(y_hbm.at[src_v.at[i]], bufs[b], gsems[b])

        def g_wait(i, b):
            pltpu.make_async_copy(y_hbm.at[src_v.at[i]], bufs[b],
                                  gsems[b]).wait()

        g_start(0, 0)
        g_start(1, 1)
        for q in range(NQ):
            sl = q % 2
            q0 = q * QC
            if q < NQ - 1:
                pltpu.async_copy(dst_hbm.at[pl.ds(base + q0 + QC, QC)],
                                 dstq[1 - sl], isem)

            @pl.loop(q0, q0 + QC - 2, step=2)
            def _(i):
                for b in range(2):
                    ii = i + b
                    g_wait(ii, b)
                    pltpu.sync_copy(bufs[b],
                                    accum.at[dstq[sl].at[ii - q0]],
                                    add=True)
                    g_start(ii + 2, b)

            for b in range(2):
                ii = q0 + QC - 2 + b
                g_wait(ii, b)
                pltpu.sync_copy(bufs[b], accum.at[dstq[sl].at[ii - q0]],
                                add=True)
                if q < NQ - 1:
                    g_start(ii + 2, b)

            if q < NQ - 1:
                pltpu.make_async_copy(dst_hbm.at[pl.ds(base + q0 + QC, QC)],
                                      dstq[1 - sl], isem).wait()

        plsc.subcore_barrier()
        pltpu.sync_copy(accum.at[pl.ds(s * RPT, RPT)],
                        out_hbm.at[c, pl.ds(s * RPT, RPT)])

    return k(y, src2d, dst2d)


def _tc_mm(x_pad, w, hist_t):
    def body(x_ref, w_ref, h_ref, y_ref, xw_ref):
        deg = h_ref[:, 0:1] + h_ref[:, 1:2] + 1.0
        dis = lax.rsqrt(deg)
        xw = jnp.dot(x_ref[...], w_ref[...],
                     preferred_element_type=jnp.float32,
                     precision=lax.Precision.HIGHEST)
        xw_ref[...] = xw
        y_ref[...] = xw * dis

    blk = 2048
    return pl.pallas_call(
        body,
        grid=(NPAD // blk,),
        in_specs=[pl.BlockSpec((blk, D), lambda i: (i, 0)),
                  pl.BlockSpec((D, D), lambda i: (0, 0)),
                  pl.BlockSpec((blk, NC), lambda i: (i, 0))],
        out_specs=(pl.BlockSpec((blk, D), lambda i: (i, 0)),
                   pl.BlockSpec((blk, D), lambda i: (i, 0))),
        out_shape=(jax.ShapeDtypeStruct((NPAD, D), jnp.float32),
                   jax.ShapeDtypeStruct((NPAD, D), jnp.float32)),
    )(x_pad, w, hist_t)


def _tc_final(partials, hist_t, xw, b2d):
    def body(p_ref, h_ref, xw_ref, b_ref, o_ref):
        deg = h_ref[:, 0:1] + h_ref[:, 1:2] + 1.0
        dis = lax.rsqrt(deg)
        agg = p_ref[0] + p_ref[1]
        o_ref[...] = agg * dis + xw_ref[...] * (dis * dis) + b_ref[...]

    blk = 2000
    return pl.pallas_call(
        body,
        grid=(N // blk,),
        in_specs=[pl.BlockSpec((NC, blk, D), lambda i: (0, i, 0)),
                  pl.BlockSpec((blk, NC), lambda i: (i, 0)),
                  pl.BlockSpec((blk, D), lambda i: (i, 0)),
                  pl.BlockSpec((1, D), lambda i: (0, 0))],
        out_specs=pl.BlockSpec((blk, D), lambda i: (i, 0)),
        out_shape=jax.ShapeDtypeStruct((N, D), jnp.float32),
    )(partials, hist_t, xw, b2d)


def kernel(x, edge_index, W, b):
    src_p = edge_index[0].reshape(NW * CH, CW)
    dst_p = edge_index[1].reshape(NW * CH, CW)
    x_pad = jnp.pad(x, ((0, NPAD - N), (0, 0)))

    hist = _sc_hist(dst_p)
    hist_t = hist.T
    y, xw = _tc_mm(x_pad, W, hist_t)
    part = _sc_msg(y, src_p, dst_p)
    return _tc_final(part, hist_t, xw, b.reshape(1, D))


# speedup vs baseline: 1.8590x; 1.0474x over previous
import dataclasses
import functools

import numpy as np

import jax
import jax.numpy as jnp
from jax import lax
from jax.experimental import pallas as pl
from jax.experimental.pallas import tpu as pltpu
from jax.experimental.pallas import tpu_sc as plsc

N = 10000
E = 320000
D = 128
NPAD = 10240
NC, NS, L = 2, 16, 16
NW = NC * NS
CW = 125
CH = 80
RPT = NPAD // NS


def _mesh():
    return plsc.VectorSubcoreMesh(core_axis_name="c", subcore_axis_name="s")


def _sc_params(**kw):
    cp = pltpu.CompilerParams(**kw)
    if "needs_layout_passes" in pltpu.CompilerParams.__dataclass_fields__:
        cp = dataclasses.replace(cp, needs_layout_passes=False)
    return cp


HIST_WIN = 16


def _sc_hist(dst2d):
    @functools.partial(
        pl.kernel,
        out_type=jax.ShapeDtypeStruct((NC, NPAD), jnp.float32),
        mesh=_mesh(),
        scratch_types=[
            pltpu.VMEM((CH, CW), jnp.int32),
            pltpu.VMEM((128,), jnp.float32),
            pltpu.VMEM((RPT,), jnp.float32),
            pltpu.VMEM_SHARED((NPAD,), jnp.float32),
            pltpu.SemaphoreType.DMA,
        ],
    )
    def k(dst_hbm, out_hbm, dst_v, ones_v, zv, hist_sh, sem):
        c = lax.axis_index("c")
        s = lax.axis_index("s")
        w = c * NS + s
        pltpu.sync_copy(dst_hbm.at[pl.ds(w * CH, CH)], dst_v)

        @pl.loop(0, 8)
        def _(i):
            ones_v[pl.ds(i * L, L)] = jnp.ones((L,), jnp.float32)

        @pl.loop(0, RPT // L)
        def _(i):
            zv[pl.ds(i * L, L)] = jnp.zeros((L,), jnp.float32)

        pltpu.sync_copy(zv, hist_sh.at[pl.ds(s * RPT, RPT)])
        plsc.subcore_barrier()

        ones_s = ones_v.at[pl.ds(0, CW)]

        @pl.loop(0, CH)
        def _(j):
            pltpu.async_copy(ones_s, hist_sh.at[dst_v.at[j]], sem, add=True)

            @pl.when(j >= HIST_WIN)
            def _():
                pltpu.make_async_copy(
                    ones_s, hist_sh.at[dst_v.at[j - HIST_WIN]], sem).wait()

        @pl.loop(CH - HIST_WIN, CH)
        def _(j):
            pltpu.make_async_copy(ones_s, hist_sh.at[dst_v.at[j]], sem).wait()

        plsc.subcore_barrier()
        pltpu.sync_copy(hist_sh.at[pl.ds(s * RPT, RPT)],
                        out_hbm.at[c, pl.ds(s * RPT, RPT)])

    return k(dst2d)


NQ = 5
QC = CH // NQ


def _sc_msg(y, src2d, dst2d):
    @functools.partial(
        pl.kernel,
        out_type=jax.ShapeDtypeStruct((NC, NPAD, D), jnp.float32),
        mesh=_mesh(),
        scratch_types=[
            pltpu.VMEM((CH, CW), jnp.int32),
            [pltpu.VMEM((QC, CW), jnp.int32)] * 2,
            [pltpu.VMEM((CW, D), jnp.float32)] * 2,
            pltpu.VMEM((16, D), jnp.float32),
            pltpu.VMEM_SHARED((NPAD, D), jnp.float32),
            [pltpu.SemaphoreType.DMA] * 2,
            pltpu.SemaphoreType.DMA,
        ],
    )
    def k(y_hbm, src_hbm, dst_hbm, out_hbm, src_v, dstq, bufs, zb, accum,
          gsems, isem):
        c = lax.axis_index("c")
        s = lax.axis_index("s")
        w = c * NS + s
        base = w * CH

        pltpu.async_copy(src_hbm.at[pl.ds(base, CH)], src_v, isem)
        pltpu.async_copy(dst_hbm.at[pl.ds(base, QC)], dstq[0], isem)

        @pl.loop(0, 16)
        def _(i):
            @pl.loop(0, D // L)
            def _(q):
                zb[i, pl.ds(q * L, L)] = jnp.zeros((L,), jnp.float32)

        @pl.loop(0, RPT // 16)
        def _(r):
            pltpu.async_copy(zb, accum.at[pl.ds(s * RPT + r * 16, 16)],
                             gsems[0])

        @pl.loop(0, RPT // 16)
        def _(r):
            pltpu.make_async_copy(zb, accum.at[pl.ds(s * RPT + r * 16, 16)],
                                  gsems[0]).wait()

        pltpu.make_async_copy(src_hbm.at[pl.ds(base, CH)], src_v,
                              isem).wait()
        pltpu.make_async_copy(dst_hbm.at[pl.ds(base, QC)], dstq[0],
                              isem).wait()
        plsc.subcore_barrier()

        def g_start(i, b):
            pltpu.async_copy(y_hbm.at[src_v.at[i]], bufs[b], gsems[b])

        def g_wait(i, b):
            pltpu.make_async_copy(y_hbm.at[src_v.at[i]], bufs[b],
                                  gsems[b]).wait()

        g_start(0, 0)
        g_start(1, 1)
        for q in range(NQ):
            sl = q % 2
            q0 = q * QC
            if q < NQ - 1:
                pltpu.async_copy(dst_hbm.at[pl.ds(base + q0 + QC, QC)],
                                 dstq[1 - sl], isem)

            @pl.loop(q0, q0 + QC - 2, step=2)
            def _(i):
                for b in range(2):
                    ii = i + b
                    g_wait(ii, b)
                    pltpu.sync_copy(bufs[b],
                                    accum.at[dstq[sl].at[ii - q0]],
                                    add=True)
                    g_start(ii + 2, b)

            for b in range(2):
                ii = q0 + QC - 2 + b
                g_wait(ii, b)
                pltpu.sync_copy(bufs[b], accum.at[dstq[sl].at[ii - q0]],
                                add=True)
                if q < NQ - 1:
                    g_start(ii + 2, b)

            if q < NQ - 1:
                pltpu.make_async_copy(dst_hbm.at[pl.ds(base + q0 + QC, QC)],
                                      dstq[1 - sl], isem).wait()

        plsc.subcore_barrier()
        pltpu.sync_copy(accum.at[pl.ds(s * RPT, RPT)],
                        out_hbm.at[c, pl.ds(s * RPT, RPT)])

    return k(y, src2d, dst2d)


def _tc_mm(x, w, hist_t):
    def body(x_ref, w_ref, h_ref, y_ref, xw_ref):
        deg = h_ref[:, 0:1] + h_ref[:, 1:2] + 1.0
        dis = lax.rsqrt(deg)
        xw = jnp.dot(x_ref[...], w_ref[...],
                     preferred_element_type=jnp.float32,
                     precision=lax.Precision.HIGHEST)
        xw_ref[...] = xw
        y_ref[...] = xw * dis

    blk = 2000
    return pl.pallas_call(
        body,
        grid=(N // blk,),
        in_specs=[pl.BlockSpec((blk, D), lambda i: (i, 0)),
                  pl.BlockSpec((D, D), lambda i: (0, 0)),
                  pl.BlockSpec((blk, NC), lambda i: (i, 0))],
        out_specs=(pl.BlockSpec((blk, D), lambda i: (i, 0)),
                   pl.BlockSpec((blk, D), lambda i: (i, 0))),
        out_shape=(jax.ShapeDtypeStruct((NPAD, D), jnp.float32),
                   jax.ShapeDtypeStruct((NPAD, D), jnp.float32)),
    )(x, w, hist_t)


def _tc_final(partials, hist_t, xw, b2d):
    def body(p_ref, h_ref, xw_ref, b_ref, o_ref):
        deg = h_ref[:, 0:1] + h_ref[:, 1:2] + 1.0
        dis = lax.rsqrt(deg)
        agg = p_ref[0] + p_ref[1]
        o_ref[...] = agg * dis + xw_ref[...] * (dis * dis) + b_ref[...]

    blk = 2000
    return pl.pallas_call(
        body,
        grid=(N // blk,),
        in_specs=[pl.BlockSpec((NC, blk, D), lambda i: (0, i, 0)),
                  pl.BlockSpec((blk, NC), lambda i: (i, 0)),
                  pl.BlockSpec((blk, D), lambda i: (i, 0)),
                  pl.BlockSpec((1, D), lambda i: (0, 0))],
        out_specs=pl.BlockSpec((blk, D), lambda i: (i, 0)),
        out_shape=jax.ShapeDtypeStruct((N, D), jnp.float32),
    )(partials, hist_t, xw, b2d)


def kernel(x, edge_index, W, b):
    src_p = edge_index[0].reshape(NW * CH, CW)
    dst_p = edge_index[1].reshape(NW * CH, CW)

    hist = _sc_hist(dst_p)
    hist_t = hist.T
    y, xw = _tc_mm(x, W, hist_t)
    part = _sc_msg(y, src_p, dst_p)
    return _tc_final(part, hist_t, xw, b.reshape(1, D))
